# baseline (speedup 1.0000x reference)
import functools

import jax
import jax.numpy as jnp
from jax import lax
from jax.experimental import pallas as pl
from jax.experimental.pallas import tpu as pltpu
from jax.experimental.pallas import tpu_sc as plsc

B = 16384
D = 64
NC = 2
NS = 16
NW = NC * NS
BPW = B // NW
CHUNK = 128
N_CHUNKS = BPW // CHUNK
NBUF = 2
LANES = 16
GROUPS = CHUNK // LANES


def _mf_body(user_hbm, anime_hbm, uf_hbm, af_hbm, out_hbm,
             uidx, aidx, urow_idx, arow_idx, urows, arows, outv, sems):
    wid = lax.axis_index("s") * NC + lax.axis_index("c")
    base = pl.multiple_of(wid * BPW, BPW)

    for k in range(N_CHUNKS):
        pltpu.sync_copy(user_hbm.at[pl.ds(base + k * CHUNK, CHUNK)],
                        uidx.at[k])
        pltpu.sync_copy(anime_hbm.at[pl.ds(base + k * CHUNK, CHUNK)],
                        aidx.at[k])

    for k in range(N_CHUNKS):
        for g in range(GROUPS):
            sl = pl.ds(g * LANES, LANES)
            urow_idx[k, sl] = jnp.right_shift(uidx[k, sl], 1)
            arow_idx[k, sl] = jnp.right_shift(aidx[k, sl], 1)

    def start_chunk(k):
        buf = k % NBUF
        cu = pltpu.async_copy(uf_hbm.at[urow_idx.at[k]], urows.at[buf],
                              sems.at[buf, 0])
        ca = pltpu.async_copy(af_hbm.at[arow_idx.at[k]], arows.at[buf],
                              sems.at[buf, 1])
        return cu, ca

    inflight = [start_chunk(k) for k in range(NBUF)]

    iota = lax.iota(jnp.int32, LANES)
    gather_dnums = lax.GatherDimensionNumbers(
        offset_dims=(), collapsed_slice_dims=(0,), start_index_map=(0,))
    rot_idx = [jnp.bitwise_and(iota + r, LANES - 1) for r in (8, 4, 2, 1)]

    def rot(x, ridx):
        return lax.gather(x, ridx[:, None], dimension_numbers=gather_dnums,
                          slice_sizes=(1,),
                          mode=lax.GatherScatterMode.PROMISE_IN_BOUNDS)

    for k in range(N_CHUNKS):
        buf = k % NBUF
        cu, ca = inflight[buf]
        cu.wait()
        ca.wait()

        def group(g, carry, k=k, buf=buf):
            gbase = pl.multiple_of(g * LANES, LANES)
            outvec = jnp.zeros((LANES,), jnp.float32)
            upar = jnp.bitwise_and(uidx[k, pl.ds(gbase, LANES)], 1) * D
            apar = jnp.bitwise_and(aidx[k, pl.ds(gbase, LANES)], 1) * D
            for j in range(LANES):
                row = gbase + j
                uoff = pl.multiple_of(upar[j], D)
                aoff = pl.multiple_of(apar[j], D)
                acc = (urows[buf, row, pl.ds(uoff, LANES)] *
                       arows[buf, row, pl.ds(aoff, LANES)])
                for c in range(1, D // LANES):
                    acc = acc + (
                        urows[buf, row, pl.ds(uoff + c * LANES, LANES)] *
                        arows[buf, row, pl.ds(aoff + c * LANES, LANES)])
                for ridx in rot_idx:
                    acc = acc + rot(acc, ridx)
                outvec = jnp.where(iota == j, acc, outvec)
            outv[pl.ds(pl.multiple_of(k * CHUNK, CHUNK) + gbase, LANES)] = (
                outvec)
            return carry

        lax.fori_loop(0, GROUPS, group, 0)

        if k + NBUF < N_CHUNKS:
            inflight[buf] = start_chunk(k + NBUF)

    pltpu.sync_copy(outv, out_hbm.at[pl.ds(base, BPW)])


_mf_kernel = functools.partial(
    pl.kernel,
    out_type=jax.ShapeDtypeStruct((B,), jnp.float32),
    mesh=plsc.VectorSubcoreMesh(core_axis_name="c", subcore_axis_name="s"),
    scratch_types=[
        pltpu.VMEM((N_CHUNKS, CHUNK), jnp.int32),
        pltpu.VMEM((N_CHUNKS, CHUNK), jnp.int32),
        pltpu.VMEM((N_CHUNKS, CHUNK), jnp.int32),
        pltpu.VMEM((N_CHUNKS, CHUNK), jnp.int32),
        pltpu.VMEM((NBUF, CHUNK, 2 * D), jnp.float32),
        pltpu.VMEM((NBUF, CHUNK, 2 * D), jnp.float32),
        pltpu.VMEM((BPW,), jnp.float32),
        pltpu.SemaphoreType.DMA((NBUF, 2)),
    ],
    compiler_params=pltpu.CompilerParams(use_tc_tiling_on_sc=True),
)(_mf_body)


def kernel(user, anime, user_factors, anime_factors):
    uf2 = user_factors.reshape(user_factors.shape[0] // 2, 2 * D)
    af2 = anime_factors.reshape(anime_factors.shape[0] // 2, 2 * D)
    return _mf_kernel(user.astype(jnp.int32), anime.astype(jnp.int32),
                      uf2, af2)
